# baseline (speedup 1.0000x reference)
import functools

import jax
import jax.numpy as jnp
from jax import lax
from jax.experimental import pallas as pl
from jax.experimental.pallas import tpu as pltpu
from jax.experimental.pallas import tpu_sc as plsc

_NC = 2
_NS = 16
_NW = _NC * _NS
_CHUNK = 32


def _sc_copy_body(w_hbm, out_hbm, buf, isem, osem, *, rows_per_w):
    wid = lax.axis_index("s") * _NC + lax.axis_index("c")
    base = wid * rows_per_w
    n = rows_per_w // _CHUNK

    def chunk(i):
        return pl.ds(base + i * _CHUNK, _CHUNK)

    in_cp = [
        pltpu.make_async_copy(w_hbm.at[chunk(i)], buf.at[i % 2], isem.at[i % 2])
        for i in range(n)
    ]
    out_cp = [
        pltpu.make_async_copy(buf.at[i % 2], out_hbm.at[0, chunk(i)], osem.at[i % 2])
        for i in range(n)
    ]

    in_cp[0].start()
    for i in range(n):
        if i + 1 < n:
            if i >= 1:
                out_cp[i - 1].wait()
            in_cp[i + 1].start()
        in_cp[i].wait()
        out_cp[i].start()
    if n >= 2:
        out_cp[n - 2].wait()
    out_cp[n - 1].wait()


def kernel(x, W):
    del x
    rows, dim = W.shape
    rows_per_w = rows // _NW
    mesh = plsc.VectorSubcoreMesh(core_axis_name="c", subcore_axis_name="s")
    sc_copy = functools.partial(
        pl.kernel,
        mesh=mesh,
        out_type=jax.ShapeDtypeStruct((1, rows, dim), W.dtype),
        scratch_types=[
            pltpu.VMEM((2, _CHUNK, dim), W.dtype),
            pltpu.SemaphoreType.DMA((2,)),
            pltpu.SemaphoreType.DMA((2,)),
        ],
    )(functools.partial(_sc_copy_body, rows_per_w=rows_per_w))
    return sc_copy(W)


# speedup vs baseline: 1.0117x; 1.0117x over previous
import functools

import jax
import jax.numpy as jnp
from jax import lax
from jax.experimental import pallas as pl
from jax.experimental.pallas import tpu as pltpu
from jax.experimental.pallas import tpu_sc as plsc

_NC = 2
_NS = 16
_NW = _NC * _NS
_CHUNK = 16
_NBUF = 4


def _sc_copy_body(w_hbm, out_hbm, buf, isem, osem, *, rows_per_w):
    wid = lax.axis_index("s") * _NC + lax.axis_index("c")
    base = wid * rows_per_w
    n = rows_per_w // _CHUNK

    def chunk(i):
        return pl.ds(base + i * _CHUNK, _CHUNK)

    in_cp = [
        pltpu.make_async_copy(w_hbm.at[chunk(i)], buf.at[i % _NBUF], isem.at[i % _NBUF])
        for i in range(n)
    ]
    out_cp = [
        pltpu.make_async_copy(buf.at[i % _NBUF], out_hbm.at[0, chunk(i)], osem.at[i % _NBUF])
        for i in range(n)
    ]

    for i in range(min(_NBUF, n)):
        in_cp[i].start()
    for i in range(n):
        in_cp[i].wait()
        out_cp[i].start()
        j = i + _NBUF
        if j < n:
            out_cp[i].wait()
            in_cp[j].start()
    for i in range(max(0, n - _NBUF), n):
        out_cp[i].wait()


def kernel(x, W):
    del x
    rows, dim = W.shape
    rows_per_w = rows // _NW
    mesh = plsc.VectorSubcoreMesh(core_axis_name="c", subcore_axis_name="s")
    sc_copy = functools.partial(
        pl.kernel,
        mesh=mesh,
        out_type=jax.ShapeDtypeStruct((1, rows, dim), W.dtype),
        scratch_types=[
            pltpu.VMEM((2, _CHUNK, dim), W.dtype),
            pltpu.SemaphoreType.DMA((2,)),
            pltpu.SemaphoreType.DMA((2,)),
        ],
    )(functools.partial(_sc_copy_body, rows_per_w=rows_per_w))
    return sc_copy(W)
